# baseline (speedup 1.0000x reference)
import functools

import jax
import jax.numpy as jnp
from jax import lax
from jax.experimental import pallas as pl
from jax.experimental.pallas import tpu as pltpu
from jax.experimental.pallas import tpu_sc as plsc

N = 10000
E = 320000
DF = 128
DH = 16
DC = 40
DCP = 48

NC, NS, L = 2, 16, 16
NW = NC * NS
EPW = E // NW
EPT = E // NS
NPAD = 10240
RPT = NPAD // NS

_sc_mesh = plsc.VectorSubcoreMesh(core_axis_name="c", subcore_axis_name="s")


def _newton_rsqrt(d):
    i = lax.bitcast_convert_type(d, jnp.int32)
    i = jnp.int32(0x5F3759DF) - lax.shift_right_logical(i, 1)
    y = lax.bitcast_convert_type(i, jnp.float32)
    for _ in range(3):
        y = y * (1.5 - 0.5 * d * y * y)
    return y


def _make_agg_kernel(D, chunks, fuse_deg):
    nsub = len(chunks)
    offs = [sum(chunks[:i]) for i in range(nsub)]
    C = max(chunks)
    has_tail = chunks[-1] != C
    deg_C = 2000
    deg_sub = EPT // deg_C

    out_type = [jax.ShapeDtypeStruct((NC, NPAD, D), jnp.float32)]
    scratch = [
        [pltpu.VMEM((C,), jnp.int32) for _ in range(2)],
        [pltpu.VMEM((C,), jnp.int32) for _ in range(3)],
        [pltpu.VMEM((C,), jnp.float32) for _ in range(2)],
        [pltpu.VMEM((C, D), jnp.float32) for _ in range(2)],
        pltpu.VMEM_SHARED((NPAD, D), jnp.float32),
        [pltpu.SemaphoreType.DMA for _ in range(2)],
        [pltpu.SemaphoreType.DMA for _ in range(2)],
        [pltpu.SemaphoreType.DMA for _ in range(2)],
    ]
    if has_tail:
        scratch.append(pltpu.VMEM((chunks[-1],), jnp.int32))
    if fuse_deg:
        out_type.append(jax.ShapeDtypeStruct((NPAD,), jnp.float32))
        scratch += [
            [pltpu.VMEM((C,), jnp.float32) for _ in range(2)],
            pltpu.VMEM((RPT,), jnp.float32),
            pltpu.VMEM_SHARED((NPAD,), jnp.float32),
            [pltpu.SemaphoreType.DMA for _ in range(2)],
        ]

    def body(tab_hbm, ei_hbm, ew_hbm, out_hbm, *rest):
        rest = list(rest)
        if fuse_deg:
            dinv_hbm = rest.pop(0)
        row_v, col_v, ew_v, rows_v, acc_s, isem, gsem, ssem = rest[:8]
        rest = rest[8:]
        if has_tail:
            col_t = rest.pop(0)
        if fuse_deg:
            dr_v, zbuf, deg_s, dsem = rest
        c = lax.axis_index("c")
        s = lax.axis_index("s")
        wid = s * NC + c
        base = wid * EPW

        def col_ref(k):
            if has_tail and k == nsub - 1:
                return col_t
            return col_v[k % 3]

        def rows2d(b):
            return rows_v[b]

        def fire_idx(k):
            eb = base + offs[k]
            ck = chunks[k]
            b = k % 2
            return (
                pltpu.async_copy(ei_hbm.at[0, pl.ds(eb, ck)],
                                 row_v[b].at[pl.ds(0, ck)], isem[b]),
                pltpu.async_copy(ew_hbm.at[pl.ds(eb, ck)],
                                 ew_v[b].at[pl.ds(0, ck)], isem[b]),
                pltpu.async_copy(ei_hbm.at[1, pl.ds(eb, ck)], col_ref(k),
                                 isem[b]),
            )

        def zfill(i, _):
            for j in range(-(-D // L)):
                o = min(j * L, D - L)
                rows_v[0][i, pl.ds(o, L)] = jnp.zeros((L,), jnp.float32)
            return 0

        lax.fori_loop(0, RPT, zfill, 0, unroll=4)
        pltpu.sync_copy(rows2d(0).at[pl.ds(0, RPT)],
                        acc_s.at[pl.ds(s * RPT, RPT)])

        if fuse_deg:
            row0_d = pltpu.async_copy(ei_hbm.at[0, pl.ds(base, chunks[0])],
                                      row_v[0], gsem[0])

            def zfill1(i, _):
                zbuf[pl.ds(i * L, L)] = jnp.zeros((L,), jnp.float32)
                return 0

            lax.fori_loop(0, RPT // L, zfill1, 0, unroll=4)
            pltpu.sync_copy(zbuf, deg_s.at[pl.ds(s * RPT, RPT)])
            plsc.subcore_barrier()

            row0_d.wait()
            tab0_d = pltpu.async_copy(tab_hbm.at[row_v[0]], rows_v[0],
                                      gsem[0])

            dd = [None, None]
            for k in range(deg_sub):
                b = k % 2
                eb = s * EPT + k * deg_C
                dc = pltpu.async_copy(ei_hbm.at[1, pl.ds(eb, deg_C)],
                                      col_v[k % 3], isem[b])
                de = pltpu.async_copy(ew_hbm.at[pl.ds(eb, deg_C)], ew_v[b],
                                      isem[b])
                if dd[b] is not None:
                    dd[b].wait()
                dc.wait()
                de.wait()
                dd[b] = pltpu.async_copy(ew_v[b], deg_s.at[col_v[k % 3]],
                                         ssem[b], add=True)
            for d in dd:
                d.wait()
            ew0_d = (
                pltpu.async_copy(ew_hbm.at[pl.ds(base, chunks[0])],
                                 ew_v[0], isem[0]),
                pltpu.async_copy(ei_hbm.at[1, pl.ds(base, chunks[0])],
                                 col_ref(0), isem[0]),
            )
            idx1_d = fire_idx(1) if nsub > 1 else None
            plsc.subcore_barrier()

            pltpu.sync_copy(deg_s.at[pl.ds(s * RPT, RPT)], zbuf)

            def nwt(i, _):
                d = zbuf[pl.ds(i * L, L)] + 1.0
                zbuf[pl.ds(i * L, L)] = _newton_rsqrt(d)
                return 0

            lax.fori_loop(0, RPT // L, nwt, 0)
            pltpu.sync_copy(zbuf, deg_s.at[pl.ds(s * RPT, RPT)])

            @pl.when(c == 0)
            def _():
                pltpu.sync_copy(zbuf, dinv_hbm.at[pl.ds(s * RPT, RPT)])

        plsc.subcore_barrier()

        def fire_gather(k):
            ck = chunks[k]
            b = k % 2
            idx = row_v[b] if ck == C else row_v[b].at[pl.ds(0, ck)]
            dst = rows2d(b) if ck == C else rows2d(b).at[pl.ds(0, ck)]
            g = pltpu.async_copy(tab_hbm.at[idx], dst, gsem[b])
            if fuse_deg:
                ddst = dr_v[b] if ck == C else dr_v[b].at[pl.ds(0, ck)]
                g2 = pltpu.async_copy(deg_s.at[idx], ddst, dsem[b])
                return (g, g2)
            return (g,)

        def fire_scatter(k):
            ck = chunks[k]
            b = k % 2
            src = rows2d(b) if ck == C else rows2d(b).at[pl.ds(0, ck)]
            return pltpu.async_copy(src, acc_s.at[col_ref(k)], ssem[b],
                                    add=True)

        idx_d = [None, None]
        gat_d = [None, None]
        sca_d = [None, None]
        if fuse_deg:
            idx_d[1] = idx1_d
            for d in ew0_d:
                d.wait()
            dr0_d = pltpu.async_copy(deg_s.at[row_v[0]], dr_v[0], dsem[0])
            gat_d[0] = (tab0_d, dr0_d)
        else:
            idx_d[0] = fire_idx(0)
            if nsub > 1:
                idx_d[1] = fire_idx(1)
            for d in idx_d[0]:
                d.wait()
            gat_d[0] = fire_gather(0)

        for k in range(nsub):
            b = k % 2
            for d in gat_d[b]:
                d.wait()
            if k + 1 < nsub:
                for d in idx_d[1 - b]:
                    d.wait()
                if k >= 1:
                    sca_d[1 - b].wait()
                gat_d[1 - b] = fire_gather(k + 1)

            def scale(g, _):
                ew16 = ew_v[b][pl.ds(g * L, L)]
                if fuse_deg:
                    ew16 = ew16 * dr_v[b][pl.ds(g * L, L)]
                for l in range(L):
                    w = ew16[l]
                    e = g * L + l
                    offs_j = [min(j * L, D - L) for j in range(-(-D // L))]
                    vals = [rows_v[b][e, pl.ds(o, L)] for o in offs_j]
                    for o, v in zip(offs_j, vals):
                        rows_v[b][e, pl.ds(o, L)] = v * w
                return 0

            lax.fori_loop(0, chunks[k] // L, scale, 0, unroll=2)
            sca_d[b] = fire_scatter(k)
            if k + 2 < nsub:
                idx_d[b] = fire_idx(k + 2)

        if nsub >= 2:
            sca_d[nsub % 2].wait()
        sca_d[(nsub - 1) % 2].wait()
        plsc.subcore_barrier()
        pltpu.sync_copy(acc_s.at[pl.ds(s * RPT, RPT)],
                        out_hbm.at[c, pl.ds(s * RPT, RPT)])

    return pl.kernel(
        body,
        out_type=tuple(out_type) if fuse_deg else out_type[0],
        mesh=_sc_mesh,
        compiler_params=pltpu.CompilerParams(use_tc_tiling_on_sc=False),
        scratch_types=scratch,
    )


_agg16 = _make_agg_kernel(DH, [2000] * 5, fuse_deg=True)
_agg40 = _make_agg_kernel(DC, [1008] * 9 + [928], fuse_deg=False)


def _tc1_body(x_ref, w1_ref, h1_ref):
    h1_ref[...] = jnp.dot(x_ref[...], w1_ref[...],
                          preferred_element_type=jnp.float32)


def _tc2_body(s1p_ref, h1_ref, dinv_ref, b1_ref, w2_ref, h2p_ref):
    dinv = dinv_ref[:N, :]
    t = s1p_ref[0, :N, :] + s1p_ref[1, :N, :]
    z1 = dinv * t + (dinv * dinv) * h1_ref[...] + b1_ref[...]
    a = jnp.maximum(z1, 0.0)
    h2 = jnp.dot(a, w2_ref[...], preferred_element_type=jnp.float32)
    h2p_ref[...] = dinv * h2


def _tc3_body(s2p_ref, h2p_ref, dinv_ref, b2_ref, out_ref):
    t = s2p_ref[0, :N, :] + s2p_ref[1, :N, :] + h2p_ref[...]
    z = dinv_ref[:N, :] * t + b2_ref[...]
    m = jnp.max(z, axis=1, keepdims=True)
    lse = m + jnp.log(jnp.sum(jnp.exp(z - m), axis=1, keepdims=True))
    out_ref[...] = z - lse


def kernel(x, edge_index, edge_weight, W1, b1, W2, b2):
    ei = edge_index.astype(jnp.int32)
    ew = edge_weight.astype(jnp.float32)

    h1 = pl.pallas_call(
        _tc1_body,
        out_shape=jax.ShapeDtypeStruct((N, DH), jnp.float32),
    )(x, W1)

    s1p, dinv = _agg16(h1, ei, ew)
    dinv = dinv.reshape(NPAD, 1)

    h2p = pl.pallas_call(
        _tc2_body,
        out_shape=jax.ShapeDtypeStruct((N, DC), jnp.float32),
    )(s1p, h1, dinv, b1.reshape(1, DH), W2)

    s2p = _agg40(h2p, ei, ew)

    out = pl.pallas_call(
        _tc3_body,
        out_shape=jax.ShapeDtypeStruct((N, DC), jnp.float32),
    )(s2p, h2p, dinv, b2.reshape(1, DC))
    return out


# speedup vs baseline: 1.1465x; 1.1465x over previous
import functools

import jax
import jax.numpy as jnp
from jax import lax
from jax.experimental import pallas as pl
from jax.experimental.pallas import tpu as pltpu
from jax.experimental.pallas import tpu_sc as plsc

N = 10000
E = 320000
DF = 128
DH = 16
DC = 40
DCP = 48

NC, NS, L = 2, 16, 16
NW = NC * NS
EPW = E // NW
EPT = E // NS
NPAD = 10240
RPT = NPAD // NS

_sc_mesh = plsc.VectorSubcoreMesh(core_axis_name="c", subcore_axis_name="s")


def _newton_rsqrt(d):
    i = lax.bitcast_convert_type(d, jnp.int32)
    i = jnp.int32(0x5F3759DF) - lax.shift_right_logical(i, 1)
    y = lax.bitcast_convert_type(i, jnp.float32)
    for _ in range(3):
        y = y * (1.5 - 0.5 * d * y * y)
    return y


def _make_agg_kernel(D, chunks, fuse_deg, stage_tab=False):
    nsub = len(chunks)
    offs = [sum(chunks[:i]) for i in range(nsub)]
    C = max(chunks)
    has_tail = chunks[-1] != C
    deg_C = 2000
    deg_sub = EPT // deg_C

    out_type = [jax.ShapeDtypeStruct((NC, NPAD, D), jnp.float32)]
    scratch = [
        [pltpu.VMEM((C,), jnp.int32) for _ in range(2)],
        [pltpu.VMEM((C,), jnp.int32) for _ in range(3)],
        [pltpu.VMEM((C,), jnp.float32) for _ in range(2)],
        [pltpu.VMEM((C, D), jnp.float32) for _ in range(2)],
        pltpu.VMEM_SHARED((NPAD, D), jnp.float32),
        [pltpu.SemaphoreType.DMA for _ in range(2)],
        [pltpu.SemaphoreType.DMA for _ in range(2)],
        [pltpu.SemaphoreType.DMA for _ in range(2)],
    ]
    if has_tail:
        scratch.append(pltpu.VMEM((chunks[-1],), jnp.int32))
    if stage_tab:
        scratch.append(pltpu.VMEM_SHARED((N, D), jnp.float32))
    if fuse_deg:
        out_type.append(jax.ShapeDtypeStruct((NPAD,), jnp.float32))
        scratch += [
            [pltpu.VMEM((C,), jnp.float32) for _ in range(2)],
            pltpu.VMEM((RPT,), jnp.float32),
            pltpu.VMEM_SHARED((NPAD,), jnp.float32),
            [pltpu.SemaphoreType.DMA for _ in range(2)],
        ]

    def body(tab_hbm, ei_hbm, ew_hbm, out_hbm, *rest):
        rest = list(rest)
        if fuse_deg:
            dinv_hbm = rest.pop(0)
        row_v, col_v, ew_v, rows_v, acc_s, isem, gsem, ssem = rest[:8]
        rest = rest[8:]
        if has_tail:
            col_t = rest.pop(0)
        if stage_tab:
            tab_s = rest.pop(0)
        if fuse_deg:
            dr_v, zbuf, deg_s, dsem = rest
        c = lax.axis_index("c")
        s = lax.axis_index("s")
        wid = s * NC + c
        base = wid * EPW

        def col_ref(k):
            if has_tail and k == nsub - 1:
                return col_t
            return col_v[k % 3]

        def rows2d(b):
            return rows_v[b]

        def fire_idx(k):
            eb = base + offs[k]
            ck = chunks[k]
            b = k % 2
            return (
                pltpu.async_copy(ei_hbm.at[0, pl.ds(eb, ck)],
                                 row_v[b].at[pl.ds(0, ck)], isem[b]),
                pltpu.async_copy(ew_hbm.at[pl.ds(eb, ck)],
                                 ew_v[b].at[pl.ds(0, ck)], isem[b]),
                pltpu.async_copy(ei_hbm.at[1, pl.ds(eb, ck)], col_ref(k),
                                 isem[b]),
            )

        if stage_tab:
            tsl = -(-N // NS) // 8 * 8
            last = N - (NS - 1) * tsl

            @pl.when(s < NS - 1)
            def _():
                pltpu.sync_copy(tab_hbm.at[pl.ds(s * tsl, tsl)],
                                tab_s.at[pl.ds(s * tsl, tsl)])

            @pl.when(s == NS - 1)
            def _():
                pltpu.sync_copy(tab_hbm.at[pl.ds((NS - 1) * tsl, last)],
                                tab_s.at[pl.ds((NS - 1) * tsl, last)])

        def zfill(i, _):
            for j in range(-(-D // L)):
                o = min(j * L, D - L)
                rows_v[0][i, pl.ds(o, L)] = jnp.zeros((L,), jnp.float32)
            return 0

        lax.fori_loop(0, RPT, zfill, 0, unroll=4)
        pltpu.sync_copy(rows2d(0).at[pl.ds(0, RPT)],
                        acc_s.at[pl.ds(s * RPT, RPT)])

        if fuse_deg:
            row0_d = pltpu.async_copy(ei_hbm.at[0, pl.ds(base, chunks[0])],
                                      row_v[0], gsem[0])

            def zfill1(i, _):
                zbuf[pl.ds(i * L, L)] = jnp.zeros((L,), jnp.float32)
                return 0

            lax.fori_loop(0, RPT // L, zfill1, 0, unroll=4)
            pltpu.sync_copy(zbuf, deg_s.at[pl.ds(s * RPT, RPT)])
            plsc.subcore_barrier()

            row0_d.wait()
            tab0_d = pltpu.async_copy(tab_hbm.at[row_v[0]], rows_v[0],
                                      gsem[0])

            dd = [None, None]
            for k in range(deg_sub):
                b = k % 2
                eb = s * EPT + k * deg_C
                dc = pltpu.async_copy(ei_hbm.at[1, pl.ds(eb, deg_C)],
                                      col_v[k % 3], isem[b])
                de = pltpu.async_copy(ew_hbm.at[pl.ds(eb, deg_C)], ew_v[b],
                                      isem[b])
                if dd[b] is not None:
                    dd[b].wait()
                dc.wait()
                de.wait()
                dd[b] = pltpu.async_copy(ew_v[b], deg_s.at[col_v[k % 3]],
                                         ssem[b], add=True)
            for d in dd:
                d.wait()
            ew0_d = (
                pltpu.async_copy(ew_hbm.at[pl.ds(base, chunks[0])],
                                 ew_v[0], isem[0]),
                pltpu.async_copy(ei_hbm.at[1, pl.ds(base, chunks[0])],
                                 col_ref(0), isem[0]),
            )
            idx1_d = fire_idx(1) if nsub > 1 else None
            plsc.subcore_barrier()

            pltpu.sync_copy(deg_s.at[pl.ds(s * RPT, RPT)], zbuf)

            def nwt(i, _):
                d = zbuf[pl.ds(i * L, L)] + 1.0
                zbuf[pl.ds(i * L, L)] = _newton_rsqrt(d)
                return 0

            lax.fori_loop(0, RPT // L, nwt, 0)
            pltpu.sync_copy(zbuf, deg_s.at[pl.ds(s * RPT, RPT)])

            @pl.when(c == 0)
            def _():
                pltpu.sync_copy(zbuf, dinv_hbm.at[pl.ds(s * RPT, RPT)])

        plsc.subcore_barrier()

        def fire_gather(k):
            ck = chunks[k]
            b = k % 2
            idx = row_v[b] if ck == C else row_v[b].at[pl.ds(0, ck)]
            dst = rows2d(b) if ck == C else rows2d(b).at[pl.ds(0, ck)]
            src = tab_s if stage_tab else tab_hbm
            g = pltpu.async_copy(src.at[idx], dst, gsem[b])
            if fuse_deg:
                ddst = dr_v[b] if ck == C else dr_v[b].at[pl.ds(0, ck)]
                g2 = pltpu.async_copy(deg_s.at[idx], ddst, dsem[b])
                return (g, g2)
            return (g,)

        def fire_scatter(k):
            ck = chunks[k]
            b = k % 2
            src = rows2d(b) if ck == C else rows2d(b).at[pl.ds(0, ck)]
            return pltpu.async_copy(src, acc_s.at[col_ref(k)], ssem[b],
                                    add=True)

        idx_d = [None, None]
        gat_d = [None, None]
        sca_d = [None, None]
        if fuse_deg:
            idx_d[1] = idx1_d
            for d in ew0_d:
                d.wait()
            dr0_d = pltpu.async_copy(deg_s.at[row_v[0]], dr_v[0], dsem[0])
            gat_d[0] = (tab0_d, dr0_d)
        else:
            idx_d[0] = fire_idx(0)
            if nsub > 1:
                idx_d[1] = fire_idx(1)
            for d in idx_d[0]:
                d.wait()
            gat_d[0] = fire_gather(0)

        for k in range(nsub):
            b = k % 2
            for d in gat_d[b]:
                d.wait()
            if k + 1 < nsub:
                for d in idx_d[1 - b]:
                    d.wait()
                if k >= 1:
                    sca_d[1 - b].wait()
                gat_d[1 - b] = fire_gather(k + 1)

            def scale(g, _):
                ew16 = ew_v[b][pl.ds(g * L, L)]
                if fuse_deg:
                    ew16 = ew16 * dr_v[b][pl.ds(g * L, L)]
                for l in range(L):
                    w = ew16[l]
                    e = g * L + l
                    offs_j = [min(j * L, D - L) for j in range(-(-D // L))]
                    vals = [rows_v[b][e, pl.ds(o, L)] for o in offs_j]
                    for o, v in zip(offs_j, vals):
                        rows_v[b][e, pl.ds(o, L)] = v * w
                return 0

            lax.fori_loop(0, chunks[k] // L, scale, 0)
            sca_d[b] = fire_scatter(k)
            if k + 2 < nsub:
                idx_d[b] = fire_idx(k + 2)

        if nsub >= 2:
            sca_d[nsub % 2].wait()
        sca_d[(nsub - 1) % 2].wait()
        plsc.subcore_barrier()
        pltpu.sync_copy(acc_s.at[pl.ds(s * RPT, RPT)],
                        out_hbm.at[c, pl.ds(s * RPT, RPT)])

    return pl.kernel(
        body,
        out_type=tuple(out_type) if fuse_deg else out_type[0],
        mesh=_sc_mesh,
        compiler_params=pltpu.CompilerParams(use_tc_tiling_on_sc=False),
        scratch_types=scratch,
    )


_agg16 = _make_agg_kernel(DH, [2000] * 5, fuse_deg=True)
_agg40 = _make_agg_kernel(DC, [896] * 11 + [144], fuse_deg=False,
                          stage_tab=True)


def _tc1_body(x_ref, w1_ref, h1_ref):
    h1_ref[...] = jnp.dot(x_ref[...], w1_ref[...],
                          preferred_element_type=jnp.float32)


def _tc2_body(s1p_ref, h1_ref, dinv_ref, b1_ref, w2_ref, h2p_ref):
    dinv = dinv_ref[:N, :]
    t = s1p_ref[0, :N, :] + s1p_ref[1, :N, :]
    z1 = dinv * t + (dinv * dinv) * h1_ref[...] + b1_ref[...]
    a = jnp.maximum(z1, 0.0)
    h2 = jnp.dot(a, w2_ref[...], preferred_element_type=jnp.float32)
    h2p_ref[...] = dinv * h2


def _tc3_body(s2p_ref, h2p_ref, dinv_ref, b2_ref, out_ref):
    t = s2p_ref[0, :N, :] + s2p_ref[1, :N, :] + h2p_ref[...]
    z = dinv_ref[:N, :] * t + b2_ref[...]
    m = jnp.max(z, axis=1, keepdims=True)
    lse = m + jnp.log(jnp.sum(jnp.exp(z - m), axis=1, keepdims=True))
    out_ref[...] = z - lse


def kernel(x, edge_index, edge_weight, W1, b1, W2, b2):
    ei = edge_index.astype(jnp.int32)
    ew = edge_weight.astype(jnp.float32)

    h1 = pl.pallas_call(
        _tc1_body,
        out_shape=jax.ShapeDtypeStruct((N, DH), jnp.float32),
    )(x, W1)

    s1p, dinv = _agg16(h1, ei, ew)
    dinv = dinv.reshape(NPAD, 1)

    h2p = pl.pallas_call(
        _tc2_body,
        out_shape=jax.ShapeDtypeStruct((N, DC), jnp.float32),
    )(s1p, h1, dinv, b1.reshape(1, DH), W2)

    s2p = _agg40(h2p, ei, ew)

    out = pl.pallas_call(
        _tc3_body,
        out_shape=jax.ShapeDtypeStruct((N, DC), jnp.float32),
    )(s2p, h2p, dinv, b2.reshape(1, DC))
    return out


# speedup vs baseline: 1.1799x; 1.0291x over previous
import functools

import jax
import jax.numpy as jnp
from jax import lax
from jax.experimental import pallas as pl
from jax.experimental.pallas import tpu as pltpu
from jax.experimental.pallas import tpu_sc as plsc

N = 10000
E = 320000
DF = 128
DH = 16
DC = 40

NC, NS, L = 2, 16, 16
NW = NC * NS
EPW = E // NW
EPT = E // NS
NPAD = 10240
RPT = NPAD // NS

_sc_mesh = plsc.VectorSubcoreMesh(core_axis_name="c", subcore_axis_name="s")


def _newton_rsqrt(d):
    i = lax.bitcast_convert_type(d, jnp.int32)
    i = jnp.int32(0x5F3759DF) - lax.shift_right_logical(i, 1)
    y = lax.bitcast_convert_type(i, jnp.float32)
    for _ in range(3):
        y = y * (1.5 - 0.5 * d * y * y)
    return y


def _make_agg_kernel(D, chunks, fuse_deg):
    nsub = len(chunks)
    offs = [sum(chunks[:i]) for i in range(nsub)]
    C = max(chunks)
    has_tail = chunks[-1] != C
    deg_C = 2000
    deg_sub = EPT // deg_C

    out_type = [jax.ShapeDtypeStruct((NC, NPAD, D), jnp.float32)]
    scratch = [
        [pltpu.VMEM((C,), jnp.int32) for _ in range(2)],
        [pltpu.VMEM((C,), jnp.int32) for _ in range(3)],
        [pltpu.VMEM((C,), jnp.float32) for _ in range(2)],
        [pltpu.VMEM((C, D), jnp.float32) for _ in range(2)],
        pltpu.VMEM_SHARED((NPAD, D), jnp.float32),
        [pltpu.SemaphoreType.DMA for _ in range(2)],
        [pltpu.SemaphoreType.DMA for _ in range(2)],
        [pltpu.SemaphoreType.DMA for _ in range(2)],
    ]
    if has_tail:
        scratch.append(pltpu.VMEM((chunks[-1],), jnp.int32))
    if fuse_deg:
        out_type.append(jax.ShapeDtypeStruct((NPAD,), jnp.float32))
        scratch += [
            [pltpu.VMEM((C,), jnp.float32) for _ in range(2)],
            pltpu.VMEM((RPT,), jnp.float32),
            pltpu.VMEM_SHARED((NPAD,), jnp.float32),
            [pltpu.SemaphoreType.DMA for _ in range(2)],
        ]

    def body(tab_hbm, ei_hbm, ew_hbm, out_hbm, *rest):
        rest = list(rest)
        if fuse_deg:
            dinv_hbm = rest.pop(0)
        row_v, col_v, ew_v, rows_v, acc_s, isem, gsem, ssem = rest[:8]
        rest = rest[8:]
        if has_tail:
            col_t = rest.pop(0)
        if fuse_deg:
            dr_v, zbuf, deg_s, dsem = rest
        c = lax.axis_index("c")
        s = lax.axis_index("s")
        wid = s * NC + c
        base = wid * EPW

        def col_ref(k):
            if has_tail and k == nsub - 1:
                return col_t
            return col_v[k % 3]

        def rows2d(b):
            return rows_v[b]

        def fire_idx(k):
            eb = base + offs[k]
            ck = chunks[k]
            b = k % 2
            return (
                pltpu.async_copy(ei_hbm.at[0, pl.ds(eb, ck)],
                                 row_v[b].at[pl.ds(0, ck)], isem[b]),
                pltpu.async_copy(ew_hbm.at[pl.ds(eb, ck)],
                                 ew_v[b].at[pl.ds(0, ck)], isem[b]),
                pltpu.async_copy(ei_hbm.at[1, pl.ds(eb, ck)], col_ref(k),
                                 isem[b]),
            )

        def zfill(i, _):
            for j in range(-(-D // L)):
                o = min(j * L, D - L)
                rows_v[0][i, pl.ds(o, L)] = jnp.zeros((L,), jnp.float32)
            return 0

        lax.fori_loop(0, RPT, zfill, 0, unroll=4)
        pltpu.sync_copy(rows2d(0).at[pl.ds(0, RPT)],
                        acc_s.at[pl.ds(s * RPT, RPT)])

        if fuse_deg:
            row0_d = pltpu.async_copy(ei_hbm.at[0, pl.ds(base, chunks[0])],
                                      row_v[0], gsem[0])

            def zfill1(i, _):
                zbuf[pl.ds(i * L, L)] = jnp.zeros((L,), jnp.float32)
                return 0

            lax.fori_loop(0, RPT // L, zfill1, 0, unroll=4)
            pltpu.sync_copy(zbuf, deg_s.at[pl.ds(s * RPT, RPT)])
            plsc.subcore_barrier()

            row0_d.wait()
            tab0_d = pltpu.async_copy(tab_hbm.at[row_v[0]], rows_v[0],
                                      gsem[0])

            dd = [None, None]
            for k in range(deg_sub):
                b = k % 2
                eb = s * EPT + k * deg_C
                dc = pltpu.async_copy(ei_hbm.at[1, pl.ds(eb, deg_C)],
                                      col_v[k % 3], isem[b])
                de = pltpu.async_copy(ew_hbm.at[pl.ds(eb, deg_C)], ew_v[b],
                                      isem[b])
                if dd[b] is not None:
                    dd[b].wait()
                dc.wait()
                de.wait()
                dd[b] = pltpu.async_copy(ew_v[b], deg_s.at[col_v[k % 3]],
                                         ssem[b], add=True)
            for d in dd:
                d.wait()
            ew0_d = (
                pltpu.async_copy(ew_hbm.at[pl.ds(base, chunks[0])],
                                 ew_v[0], isem[0]),
                pltpu.async_copy(ei_hbm.at[1, pl.ds(base, chunks[0])],
                                 col_ref(0), isem[0]),
            )
            idx1_d = fire_idx(1) if nsub > 1 else None
            plsc.subcore_barrier()

            pltpu.sync_copy(deg_s.at[pl.ds(s * RPT, RPT)], zbuf)

            def nwt(i, _):
                d = zbuf[pl.ds(i * L, L)] + 1.0
                zbuf[pl.ds(i * L, L)] = _newton_rsqrt(d)
                return 0

            lax.fori_loop(0, RPT // L, nwt, 0)
            pltpu.sync_copy(zbuf, deg_s.at[pl.ds(s * RPT, RPT)])

            @pl.when(c == 0)
            def _():
                pltpu.sync_copy(zbuf, dinv_hbm.at[pl.ds(s * RPT, RPT)])

        plsc.subcore_barrier()

        def fire_gather(k):
            ck = chunks[k]
            b = k % 2
            idx = row_v[b] if ck == C else row_v[b].at[pl.ds(0, ck)]
            dst = rows2d(b) if ck == C else rows2d(b).at[pl.ds(0, ck)]
            g = pltpu.async_copy(tab_hbm.at[idx], dst, gsem[b])
            if fuse_deg:
                ddst = dr_v[b] if ck == C else dr_v[b].at[pl.ds(0, ck)]
                g2 = pltpu.async_copy(deg_s.at[idx], ddst, dsem[b])
                return (g, g2)
            return (g,)

        def fire_scatter(k):
            ck = chunks[k]
            b = k % 2
            src = rows2d(b) if ck == C else rows2d(b).at[pl.ds(0, ck)]
            return pltpu.async_copy(src, acc_s.at[col_ref(k)], ssem[b],
                                    add=True)

        idx_d = [None, None]
        gat_d = [None, None]
        sca_d = [None, None]
        if fuse_deg:
            idx_d[1] = idx1_d
            for d in ew0_d:
                d.wait()
            dr0_d = pltpu.async_copy(deg_s.at[row_v[0]], dr_v[0], dsem[0])
            gat_d[0] = (tab0_d, dr0_d)
        else:
            idx_d[0] = fire_idx(0)
            if nsub > 1:
                idx_d[1] = fire_idx(1)
            for d in idx_d[0]:
                d.wait()
            gat_d[0] = fire_gather(0)

        for k in range(nsub):
            b = k % 2
            for d in gat_d[b]:
                d.wait()
            if k + 1 < nsub:
                for d in idx_d[1 - b]:
                    d.wait()
                if k >= 1:
                    sca_d[1 - b].wait()
                gat_d[1 - b] = fire_gather(k + 1)

            def scale(g, _):
                ew16 = ew_v[b][pl.ds(g * L, L)]
                if fuse_deg:
                    ew16 = ew16 * dr_v[b][pl.ds(g * L, L)]
                for l in range(L):
                    w = ew16[l]
                    e = g * L + l
                    offs_j = [min(j * L, D - L) for j in range(-(-D // L))]
                    vals = [rows_v[b][e, pl.ds(o, L)] for o in offs_j]
                    for o, v in zip(offs_j, vals):
                        rows_v[b][e, pl.ds(o, L)] = v * w
                return 0

            lax.fori_loop(0, chunks[k] // L, scale, 0)
            sca_d[b] = fire_scatter(k)
            if k + 2 < nsub:
                idx_d[b] = fire_idx(k + 2)

        if nsub >= 2:
            sca_d[nsub % 2].wait()
        sca_d[(nsub - 1) % 2].wait()
        plsc.subcore_barrier()
        pltpu.sync_copy(acc_s.at[pl.ds(s * RPT, RPT)],
                        out_hbm.at[c, pl.ds(s * RPT, RPT)])

    return pl.kernel(
        body,
        out_type=tuple(out_type) if fuse_deg else out_type[0],
        mesh=_sc_mesh,
        compiler_params=pltpu.CompilerParams(use_tc_tiling_on_sc=False),
        scratch_types=scratch,
    )


_agg16 = _make_agg_kernel(DH, [2000] * 5, fuse_deg=True)
_agg40 = _make_agg_kernel(DC, [1008] * 9 + [928], fuse_deg=False)


def _tc1_body(x_ref, w1_ref, h1_ref):
    h1_ref[...] = jnp.dot(x_ref[...], w1_ref[...],
                          preferred_element_type=jnp.float32)


def _tc2_body(s1p_ref, h1_ref, dinv_ref, b1_ref, w2_ref, h2p_ref):
    dinv = dinv_ref[:N, :]
    t = s1p_ref[0, :N, :] + s1p_ref[1, :N, :]
    z1 = dinv * t + (dinv * dinv) * h1_ref[...] + b1_ref[...]
    a = jnp.maximum(z1, 0.0)
    h2 = jnp.dot(a, w2_ref[...], preferred_element_type=jnp.float32)
    h2p_ref[...] = dinv * h2


def _tc3_body(s2p_ref, h2p_ref, dinv_ref, b2_ref, out_ref):
    t = s2p_ref[0, :N, :] + s2p_ref[1, :N, :] + h2p_ref[...]
    z = dinv_ref[:N, :] * t + b2_ref[...]
    m = jnp.max(z, axis=1, keepdims=True)
    lse = m + jnp.log(jnp.sum(jnp.exp(z - m), axis=1, keepdims=True))
    out_ref[...] = z - lse


def kernel(x, edge_index, edge_weight, W1, b1, W2, b2):
    ei = edge_index.astype(jnp.int32)
    ew = edge_weight.astype(jnp.float32)

    h1 = pl.pallas_call(
        _tc1_body,
        out_shape=jax.ShapeDtypeStruct((N, DH), jnp.float32),
    )(x, W1)

    s1p, dinv = _agg16(h1, ei, ew)
    dinv = dinv.reshape(NPAD, 1)

    h2p = pl.pallas_call(
        _tc2_body,
        out_shape=jax.ShapeDtypeStruct((N, DC), jnp.float32),
    )(s1p, h1, dinv, b1.reshape(1, DH), W2)

    s2p = _agg40(h2p, ei, ew)

    out = pl.pallas_call(
        _tc3_body,
        out_shape=jax.ShapeDtypeStruct((N, DC), jnp.float32),
    )(s2p, h2p, dinv, b2.reshape(1, DC))
    return out
